# baseline (speedup 1.0000x reference)
import functools
import math

import jax
import jax.numpy as jnp
from jax import lax
from jax.experimental import pallas as pl
from jax.experimental.pallas import tpu as pltpu
from jax.experimental.pallas import tpu_sc as plsc

_P = 0.05
_LANES = 16
_NBINS = 65536
_ROWS_PER_CHUNK = 56
_NBUF = 4


def _c16(v, dtype=jnp.int32):
    return jnp.full((_LANES,), v, dtype)


def _splat(x, dtype=None):
    v = jnp.broadcast_to(x, (_LANES,))
    return v if dtype is None else v.astype(dtype)


def _make_sc_select(rows, chans, k, nimg):
    info = plsc.get_sparse_core_info()
    nw = info.num_cores * info.num_subcores
    assert nimg == nw, (nimg, nw)
    rpc = _ROWS_PER_CHUNK
    nchunks = rows // rpc
    assert nchunks * rpc == rows and nchunks % _NBUF == 0
    vpr = chans // _LANES
    assert vpr * _LANES == chans
    kf = float(k)

    mesh = plsc.VectorSubcoreMesh(core_axis_name="c", subcore_axis_name="s")

    @functools.partial(
        pl.kernel,
        out_type=jax.ShapeDtypeStruct((nimg * _LANES,), jnp.float32),
        mesh=mesh,
        compiler_params=pltpu.CompilerParams(needs_layout_passes=False),
        scratch_types=[
            *[pltpu.VMEM((rpc, chans), jnp.float32) for _ in range(_NBUF)],
            pltpu.VMEM((_NBINS,), jnp.float32),
            pltpu.VMEM((_LANES,), jnp.float32),
            *[pltpu.SemaphoreType.DMA for _ in range(_NBUF)],
        ],
    )
    def sc_kernel(x_hbm, out_hbm, *refs):
        bufs = refs[:_NBUF]
        hist, tbuf = refs[_NBUF], refs[_NBUF + 1]
        sems = refs[_NBUF + 2:]
        wid = lax.axis_index("s") * info.num_cores + lax.axis_index("c")
        ones = jnp.ones((_LANES,), jnp.float32)
        lanes = lax.iota(jnp.int32, _LANES)

        def zero_hist():
            @plsc.parallel_loop(0, _NBINS // _LANES, 1, unroll=8)
            def _(j):
                hist[pl.ds(j * _LANES, _LANES)] = jnp.zeros((_LANES,), jnp.float32)

        def keys_of(v):
            bi = lax.bitcast_convert_type(v, jnp.int32)
            m = lax.shift_right_arithmetic(bi, _c16(31))
            return lax.bitwise_xor(bi, lax.bitwise_or(m, _c16(-(2**31))))

        def dma(c, buf, sem):
            return pltpu.make_async_copy(
                x_hbm.at[wid, pl.ds(c * rpc, rpc), :], buf, sem
            )

        def prime():
            for u in range(_NBUF - 1):
                dma(u, bufs[u], sems[u]).start()

        def histogram_pass(prefix_bin):
            zero_hist()

            def update(key):
                hi = lax.shift_right_logical(key, _c16(16))
                if prefix_bin is None:
                    plsc.addupdate_scatter(hist, [hi], ones)
                else:
                    lo = lax.bitwise_and(key, _c16(0xFFFF))
                    msk = hi == _splat(prefix_bin)
                    plsc.addupdate_scatter(hist, [lo], ones, mask=msk)

            def process(buf):
                @plsc.parallel_loop(0, rpc, 1, unroll=2)
                def _(r):
                    for u in range(vpr):
                        update(keys_of(buf[r, pl.ds(u * _LANES, _LANES)]))

            def turn(q, _):
                for u in range(_NBUF):
                    c = q * _NBUF + u
                    nxt = c + _NBUF - 1
                    slot = (u + _NBUF - 1) % _NBUF

                    @pl.when(nxt < nchunks)
                    def _():
                        dma(nxt, bufs[slot], sems[slot]).start()

                    dma(c, bufs[u], sems[u]).wait()
                    process(bufs[u])
                return 0

            lax.fori_loop(0, nchunks // _NBUF, turn, 0)

        def select(target):
            seg_base = jnp.int32(0)
            above = jnp.float32(0.0)
            for p in (4096, 256, 16, 1):
                base_idx = _splat(seg_base) + lanes * p
                if p > 1:
                    def sb(t, acc):
                        return acc + plsc.load_gather(hist, [base_idx + _splat(t)])

                    seg = plsc.parallel_loop(
                        0, p, 1, unroll=8, carry=jnp.zeros((_LANES,), jnp.float32)
                    )(sb)
                else:
                    seg = plsc.load_gather(hist, [base_idx])
                suf = lax.rev(plsc.cumsum(lax.rev(seg, (0,))), (0,))
                cond = (_splat(above) + suf) >= _splat(jnp.float32(target))
                lsel = plsc.all_reduce_population_count(cond) - 1
                sel = lanes == lsel
                suf_sel = jnp.sum(jnp.where(sel, suf, 0.0))
                seg_sel = jnp.sum(jnp.where(sel, seg, 0.0))
                above = above + suf_sel - seg_sel
                seg_base = seg_base + jnp.max(lsel) * p
            return seg_base, above

        prime()
        histogram_pass(None)
        prime()
        b, a = select(kf)
        histogram_pass(b)
        l, _ = select(kf - a)

        key = lax.bitwise_or(lax.shift_left(b, jnp.int32(16)), l)
        kv = _splat(key)
        bits = jnp.where(
            kv < _c16(0),
            lax.bitwise_xor(kv, _c16(-(2**31))),
            lax.bitwise_not(kv),
        )
        tbuf[...] = lax.bitcast_convert_type(bits, jnp.float32)
        pltpu.sync_copy(tbuf, out_hbm.at[pl.ds(wid * _LANES, _LANES)])

    return sc_kernel


def _mask_body(t_ref, x_ref, o_ref):
    i = pl.program_id(0)
    t = t_ref[i]
    xv = x_ref[...]
    o_ref[...] = jnp.where(xv >= t, xv, 0.0)


def _tc_mask(xt, thr, nblocks):
    nimg, rows, chans = xt.shape
    blk = rows // nblocks
    assert blk * nblocks == rows
    grid_spec = pltpu.PrefetchScalarGridSpec(
        num_scalar_prefetch=1,
        grid=(nimg, nblocks),
        in_specs=[pl.BlockSpec((1, blk, chans), lambda i, j, s: (i, j, 0))],
        out_specs=pl.BlockSpec((1, blk, chans), lambda i, j, s: (i, j, 0)),
    )
    return pl.pallas_call(
        _mask_body,
        grid_spec=grid_spec,
        out_shape=jax.ShapeDtypeStruct(xt.shape, jnp.float32),
        compiler_params=pltpu.CompilerParams(
            dimension_semantics=("parallel", "parallel"),
        ),
    )(thr, xt)


def kernel(x):
    bs, nc, h, w = x.shape
    n = nc * h * w
    k = int(math.ceil(n * _P))
    sc_select = _make_sc_select(h * w, nc, k, bs)
    xt = jnp.transpose(x, (0, 2, 3, 1)).reshape(bs, h * w, nc)
    tvec = sc_select(xt)
    thr = tvec.reshape(bs, _LANES)[:, 0]
    out = _tc_mask(xt, thr, 1)
    return jnp.transpose(out.reshape(bs, h, w, nc), (0, 3, 1, 2))


# speedup vs baseline: 1.0669x; 1.0669x over previous
import functools
import math

import jax
import jax.numpy as jnp
from jax import lax
from jax.experimental import pallas as pl
from jax.experimental.pallas import tpu as pltpu
from jax.experimental.pallas import tpu_sc as plsc

_P = 0.05
_LANES = 16
_NBINS = 65536
_ROWS_PER_CHUNK = 112
_NBUF = 2


def _c16(v, dtype=jnp.int32):
    return jnp.full((_LANES,), v, dtype)


def _splat(x, dtype=None):
    v = jnp.broadcast_to(x, (_LANES,))
    return v if dtype is None else v.astype(dtype)


def _make_sc_select(rows, chans, k, nimg):
    info = plsc.get_sparse_core_info()
    nw = info.num_cores * info.num_subcores
    assert nimg == nw, (nimg, nw)
    rpc = _ROWS_PER_CHUNK
    nchunks = rows // rpc
    assert nchunks * rpc == rows and nchunks % _NBUF == 0
    vpr = chans // _LANES
    assert vpr * _LANES == chans
    kf = float(k)

    mesh = plsc.VectorSubcoreMesh(core_axis_name="c", subcore_axis_name="s")

    @functools.partial(
        pl.kernel,
        out_type=jax.ShapeDtypeStruct((nimg * _LANES,), jnp.float32),
        mesh=mesh,
        compiler_params=pltpu.CompilerParams(needs_layout_passes=False),
        scratch_types=[
            *[pltpu.VMEM((rpc, chans), jnp.float32) for _ in range(_NBUF)],
            pltpu.VMEM((_NBINS,), jnp.float32),
            pltpu.VMEM((_LANES,), jnp.float32),
            *[pltpu.SemaphoreType.DMA for _ in range(_NBUF)],
        ],
    )
    def sc_kernel(x_hbm, out_hbm, *refs):
        bufs = refs[:_NBUF]
        hist, tbuf = refs[_NBUF], refs[_NBUF + 1]
        sems = refs[_NBUF + 2:]
        wid = lax.axis_index("s") * info.num_cores + lax.axis_index("c")
        ones = jnp.ones((_LANES,), jnp.float32)
        lanes = lax.iota(jnp.int32, _LANES)

        def zero_hist():
            @plsc.parallel_loop(0, _NBINS // _LANES, 1, unroll=8)
            def _(j):
                hist[pl.ds(j * _LANES, _LANES)] = jnp.zeros((_LANES,), jnp.float32)

        def keys_of(v):
            bi = lax.bitcast_convert_type(v, jnp.int32)
            m = lax.shift_right_arithmetic(bi, _c16(31))
            return lax.bitwise_xor(bi, lax.bitwise_or(m, _c16(-(2**31))))

        def dma(c, buf, sem):
            return pltpu.make_async_copy(
                x_hbm.at[wid, pl.ds(c * rpc, rpc), :], buf, sem
            )

        def prime():
            for u in range(_NBUF - 1):
                dma(u, bufs[u], sems[u]).start()

        def histogram_pass(prefix_bin):
            zero_hist()

            def update(key):
                hi = lax.shift_right_logical(key, _c16(16))
                if prefix_bin is None:
                    plsc.addupdate_scatter(hist, [hi], ones)
                else:
                    lo = lax.bitwise_and(key, _c16(0xFFFF))
                    msk = hi == _splat(prefix_bin)
                    plsc.addupdate_scatter(hist, [lo], ones, mask=msk)

            def process(buf):
                @plsc.parallel_loop(0, rpc, 1, unroll=2)
                def _(r):
                    for u in range(vpr):
                        update(keys_of(buf[r, pl.ds(u * _LANES, _LANES)]))

            def turn(q, _):
                for u in range(_NBUF):
                    c = q * _NBUF + u
                    nxt = c + _NBUF - 1
                    slot = (u + _NBUF - 1) % _NBUF

                    @pl.when(nxt < nchunks)
                    def _():
                        dma(nxt, bufs[slot], sems[slot]).start()

                    dma(c, bufs[u], sems[u]).wait()
                    process(bufs[u])
                return 0

            lax.fori_loop(0, nchunks // _NBUF, turn, 0)

        def select(target):
            seg_base = jnp.int32(0)
            above = jnp.float32(0.0)
            for p in (4096, 256, 16, 1):
                base_idx = _splat(seg_base) + lanes * p
                if p > 1:
                    def sb(t, acc):
                        return acc + plsc.load_gather(hist, [base_idx + _splat(t)])

                    seg = plsc.parallel_loop(
                        0, p, 1, unroll=8, carry=jnp.zeros((_LANES,), jnp.float32)
                    )(sb)
                else:
                    seg = plsc.load_gather(hist, [base_idx])
                suf = lax.rev(plsc.cumsum(lax.rev(seg, (0,))), (0,))
                cond = (_splat(above) + suf) >= _splat(jnp.float32(target))
                lsel = plsc.all_reduce_population_count(cond) - 1
                sel = lanes == lsel
                suf_sel = jnp.sum(jnp.where(sel, suf, 0.0))
                seg_sel = jnp.sum(jnp.where(sel, seg, 0.0))
                above = above + suf_sel - seg_sel
                seg_base = seg_base + jnp.max(lsel) * p
            return seg_base, above

        prime()
        histogram_pass(None)
        prime()
        b, a = select(kf)
        histogram_pass(b)
        l, _ = select(kf - a)

        key = lax.bitwise_or(lax.shift_left(b, jnp.int32(16)), l)
        kv = _splat(key)
        bits = jnp.where(
            kv < _c16(0),
            lax.bitwise_xor(kv, _c16(-(2**31))),
            lax.bitwise_not(kv),
        )
        tbuf[...] = lax.bitcast_convert_type(bits, jnp.float32)
        pltpu.sync_copy(tbuf, out_hbm.at[pl.ds(wid * _LANES, _LANES)])

    return sc_kernel


def _mask_body(t_ref, x_ref, o_ref):
    i = pl.program_id(0)
    t = t_ref[i]
    xv = x_ref[...]
    o_ref[...] = jnp.where(xv >= t, xv, 0.0)


def _tc_mask(xt, thr, nblocks):
    nimg, rows, chans = xt.shape
    blk = rows // nblocks
    assert blk * nblocks == rows
    grid_spec = pltpu.PrefetchScalarGridSpec(
        num_scalar_prefetch=1,
        grid=(nimg, nblocks),
        in_specs=[pl.BlockSpec((1, blk, chans), lambda i, j, s: (i, j, 0))],
        out_specs=pl.BlockSpec((1, blk, chans), lambda i, j, s: (i, j, 0)),
    )
    return pl.pallas_call(
        _mask_body,
        grid_spec=grid_spec,
        out_shape=jax.ShapeDtypeStruct(xt.shape, jnp.float32),
        compiler_params=pltpu.CompilerParams(
            dimension_semantics=("parallel", "parallel"),
        ),
    )(thr, xt)


def kernel(x):
    bs, nc, h, w = x.shape
    n = nc * h * w
    k = int(math.ceil(n * _P))
    sc_select = _make_sc_select(h * w, nc, k, bs)
    xt = jnp.transpose(x, (0, 2, 3, 1)).reshape(bs, h * w, nc)
    tvec = sc_select(xt)
    thr = tvec.reshape(bs, _LANES)[:, 0]
    out = _tc_mask(xt, thr, 1)
    return jnp.transpose(out.reshape(bs, h, w, nc), (0, 3, 1, 2))
